# baseline (speedup 1.0000x reference)
import functools

import jax
import jax.numpy as jnp
from jax import lax
from jax.experimental import pallas as pl
from jax.experimental.pallas import tpu as pltpu
from jax.experimental.pallas import tpu_sc as plsc

_NC = 2
_NS = 16
_NW = _NC * _NS

_N_NODES = 10000
_D = 128
_CH = 128
_NCH = 80
_E_PAD = _NW * _NCH * _CH
_ACC_N = 10240
_STRIPE = _ACC_N // _NS
_DUMMY = _N_NODES


def _sc_body(feat, pk, out, pkv, srcb, dstb, rows, acc, gs0, gs1, ss0, ss1):
    c = lax.axis_index("c")
    s = lax.axis_index("s")
    gsems = (gs0, gs1)
    ssems = (ss0, ss1)

    @pl.when(c == 0)
    def _core0():
        zeros16 = jnp.zeros((16,), jnp.float32)

        def zrow(r, carry):
            for g in range(8):
                rows[0, r, pl.ds(g * 16, 16)] = zeros16
            return carry

        lax.fori_loop(0, _CH, zrow, 0)
        base = s * _STRIPE
        for i in range(_STRIPE // _CH):
            pltpu.sync_copy(rows.at[0], acc.at[pl.ds(base + i * _CH, _CH)])
        plsc.subcore_barrier()

        def unpack(j, b):
            for g in range(8):
                v = pkv[j, pl.ds(g * 16, 16)]
                srcb[b, pl.ds(g * 16, 16)] = v & 0xFFFF
                dstb[b, pl.ds(g * 16, 16)] = v >> 16

        def gather_start(b):
            pltpu.async_copy(feat.at[srcb.at[b]], rows.at[b], gsems[b])

        def gather_wait(b):
            pltpu.make_async_copy(feat.at[srcb.at[b]], rows.at[b], gsems[b]).wait()

        def scatter_start(b):
            pltpu.async_copy(rows.at[b], acc.at[dstb.at[b]], ssems[b], add=True)

        def scatter_wait(b):
            pltpu.make_async_copy(rows.at[b], acc.at[dstb.at[b]], ssems[b]).wait()

        for half in range(2):
            pltpu.sync_copy(pk.at[half * _NS + s], pkv)

            for b in range(2):
                unpack(b, b)
                gather_start(b)

            def body(i, carry):
                for b in range(2):
                    j = i * 2 + b
                    gather_wait(b)
                    scatter_start(b)
                    nxt = j + 2

                    @pl.when(nxt < _NCH)
                    def _():
                        scatter_wait(b)
                        unpack(nxt, b)
                        gather_start(b)

                return carry

            lax.fori_loop(0, _NCH // 2, body, 0)
            for b in range(2):
                scatter_wait(b)

        plsc.subcore_barrier()
        pltpu.sync_copy(acc.at[pl.ds(base, _STRIPE)], out.at[pl.ds(base, _STRIPE)])


@jax.jit
def _sc_aggregate(feature, pk3):
    mesh = plsc.VectorSubcoreMesh(core_axis_name="c", subcore_axis_name="s")
    k = functools.partial(
        pl.kernel,
        mesh=mesh,
        out_type=jax.ShapeDtypeStruct((_ACC_N, _D), jnp.float32),
        scratch_types=[
            pltpu.VMEM((_NCH, _CH), jnp.int32),
            pltpu.VMEM((2, _CH), jnp.int32),
            pltpu.VMEM((2, _CH), jnp.int32),
            pltpu.VMEM((2, _CH, _D), jnp.float32),
            pltpu.VMEM_SHARED((_ACC_N, _D), jnp.float32),
            pltpu.SemaphoreType.DMA,
            pltpu.SemaphoreType.DMA,
            pltpu.SemaphoreType.DMA,
            pltpu.SemaphoreType.DMA,
        ],
    )(_sc_body)
    return k(feature, pk3)


def _tc_body(p_ref, wt_ref, b_ref, o_ref):
    y = jnp.dot(p_ref[...], wt_ref[...], preferred_element_type=jnp.float32)
    o_ref[...] = jnp.maximum(y + b_ref[...], 0.0)


def _tc_linear(partial, Wt, b2):
    bm = 1000
    return pl.pallas_call(
        _tc_body,
        grid=(_N_NODES // bm,),
        in_specs=[
            pl.BlockSpec((bm, _D), lambda i: (i, 0)),
            pl.BlockSpec((_D, _D), lambda i: (0, 0)),
            pl.BlockSpec((1, _D), lambda i: (0, 0)),
        ],
        out_specs=pl.BlockSpec((bm, _D), lambda i: (i, 0)),
        out_shape=jax.ShapeDtypeStruct((_N_NODES, _D), jnp.float32),
    )(partial, Wt, b2)


def kernel(feature, edge_index, W, b):
    src = edge_index[0].astype(jnp.int32)
    dst = edge_index[1].astype(jnp.int32)
    packed = src | (dst << 16)
    pad = _E_PAD - packed.shape[0]
    pad_dst = _DUMMY + (jnp.arange(pad, dtype=jnp.int32) % (_ACC_N - _N_NODES))
    packed = jnp.concatenate([packed, pad_dst << 16])
    pk3 = packed.reshape(_NW, _NCH, _CH)
    partial = _sc_aggregate(feature, pk3)
    return _tc_linear(partial, W.T, b.reshape(1, _D))


# speedup vs baseline: 1.0007x; 1.0007x over previous
import functools

import jax
import jax.numpy as jnp
from jax import lax
from jax.experimental import pallas as pl
from jax.experimental.pallas import tpu as pltpu
from jax.experimental.pallas import tpu_sc as plsc

_NC = 2
_NS = 16
_NW = _NC * _NS

_N_NODES = 10000
_D = 128
_CH = 128
_NCH = 80
_E_PAD = _NW * _NCH * _CH
_ACC_N = 10240
_STRIPE = _ACC_N // _NS
_DUMMY = _N_NODES


def _sc_body(feat, pk, out, pkv, srcb, dstb, rows, acc, gs0, gs1, ss0, ss1):
    c = lax.axis_index("c")
    s = lax.axis_index("s")
    gsems = (gs0, gs1)
    ssems = (ss0, ss1)

    @pl.when(c == 0)
    def _core0():
        zeros16 = jnp.zeros((16,), jnp.float32)

        def zrow(r, carry):
            for g in range(8):
                rows[0, r, pl.ds(g * 16, 16)] = zeros16
            return carry

        lax.fori_loop(0, _CH, zrow, 0)
        base = s * _STRIPE
        for i in range(_STRIPE // _CH):
            pltpu.sync_copy(rows.at[0], acc.at[pl.ds(base + i * _CH, _CH)])
        plsc.subcore_barrier()

        def unpack(j, b):
            for g in range(8):
                v = pkv[j, pl.ds(g * 16, 16)]
                srcb[b, pl.ds(g * 16, 16)] = v & 0xFFFF
                dstb[b, pl.ds(g * 16, 16)] = v >> 16

        def gather_start(b):
            pltpu.async_copy(feat.at[srcb.at[b]], rows.at[b], gsems[b])

        def gather_wait(b):
            pltpu.make_async_copy(feat.at[srcb.at[b]], rows.at[b], gsems[b]).wait()

        def scatter_start(b):
            pltpu.async_copy(rows.at[b], acc.at[dstb.at[b]], ssems[b], add=True)

        def scatter_wait(b):
            pltpu.make_async_copy(rows.at[b], acc.at[dstb.at[b]], ssems[b]).wait()

        def half_body(half, carry0):
            pltpu.sync_copy(pk.at[half * _NS + s], pkv)

            for b in range(2):
                unpack(b, b)
                gather_start(b)

            def body(i, carry):
                for b in range(2):
                    j = i * 2 + b
                    gather_wait(b)
                    scatter_start(b)
                    nxt = j + 2

                    @pl.when(nxt < _NCH)
                    def _():
                        scatter_wait(b)
                        unpack(nxt, b)
                        gather_start(b)

                return carry

            lax.fori_loop(0, _NCH // 2, body, 0)
            for b in range(2):
                scatter_wait(b)
            return carry0

        lax.fori_loop(0, 2, half_body, 0)

        plsc.subcore_barrier()
        pltpu.sync_copy(acc.at[pl.ds(base, _STRIPE)], out.at[pl.ds(base, _STRIPE)])


@jax.jit
def _sc_aggregate(feature, pk3):
    mesh = plsc.VectorSubcoreMesh(core_axis_name="c", subcore_axis_name="s")
    k = functools.partial(
        pl.kernel,
        mesh=mesh,
        out_type=jax.ShapeDtypeStruct((_ACC_N, _D), jnp.float32),
        scratch_types=[
            pltpu.VMEM((_NCH, _CH), jnp.int32),
            pltpu.VMEM((2, _CH), jnp.int32),
            pltpu.VMEM((2, _CH), jnp.int32),
            pltpu.VMEM((2, _CH, _D), jnp.float32),
            pltpu.VMEM_SHARED((_ACC_N, _D), jnp.float32),
            pltpu.SemaphoreType.DMA,
            pltpu.SemaphoreType.DMA,
            pltpu.SemaphoreType.DMA,
            pltpu.SemaphoreType.DMA,
        ],
    )(_sc_body)
    return k(feature, pk3)


def _tc_body(p_ref, wt_ref, b_ref, o_ref):
    y = jnp.dot(p_ref[...], wt_ref[...], preferred_element_type=jnp.float32)
    o_ref[...] = jnp.maximum(y + b_ref[...], 0.0)


def _tc_linear(partial, Wt, b2):
    bm = 1000
    return pl.pallas_call(
        _tc_body,
        grid=(_N_NODES // bm,),
        in_specs=[
            pl.BlockSpec((bm, _D), lambda i: (i, 0)),
            pl.BlockSpec((_D, _D), lambda i: (0, 0)),
            pl.BlockSpec((1, _D), lambda i: (0, 0)),
        ],
        out_specs=pl.BlockSpec((bm, _D), lambda i: (i, 0)),
        out_shape=jax.ShapeDtypeStruct((_N_NODES, _D), jnp.float32),
    )(partial, Wt, b2)


def kernel(feature, edge_index, W, b):
    src = edge_index[0].astype(jnp.int32)
    dst = edge_index[1].astype(jnp.int32)
    packed = src | (dst << 16)
    pad = _E_PAD - packed.shape[0]
    pad_dst = _DUMMY + (jnp.arange(pad, dtype=jnp.int32) % (_ACC_N - _N_NODES))
    packed = jnp.concatenate([packed, pad_dst << 16])
    pk3 = packed.reshape(_NW, _NCH, _CH)
    partial = _sc_aggregate(feature, pk3)
    return _tc_linear(partial, W.T, b.reshape(1, _D))


# speedup vs baseline: 4.6270x; 4.6236x over previous
import functools

import jax
import jax.numpy as jnp
from jax import lax
from jax.experimental import pallas as pl
from jax.experimental.pallas import tpu as pltpu
from jax.experimental.pallas import tpu_sc as plsc

_NC = 2
_NS = 16
_NW = _NC * _NS

_N_NODES = 10000
_D = 128
_CH = 128
_NCH = 80
_E_PAD = _NW * _NCH * _CH
_ACC_N = 10240
_STRIPE = _ACC_N // _NS
_DUMMY = _N_NODES


def _sc_body(feat, pk, out, pkv, srcb, dstb, rows, acc, gs0, gs1, ss0, ss1):
    c = lax.axis_index("c")
    s = lax.axis_index("s")
    slab = c * _NS + s
    gsems = (gs0, gs1)
    ssems = (ss0, ss1)

    zeros16 = jnp.zeros((16,), jnp.float32)

    def zrow(r, carry):
        for g in range(8):
            rows[0, r, pl.ds(g * 16, 16)] = zeros16
        return carry

    lax.fori_loop(0, _CH, zrow, 0)
    base = s * _STRIPE
    for i in range(_STRIPE // _CH):
        pltpu.sync_copy(rows.at[0], acc.at[pl.ds(base + i * _CH, _CH)])
    plsc.subcore_barrier()

    pltpu.sync_copy(pk.at[slab], pkv)

    def unpack(j, b):
        for g in range(8):
            v = pkv[j, pl.ds(g * 16, 16)]
            srcb[b, pl.ds(g * 16, 16)] = v & 0xFFFF
            dstb[b, pl.ds(g * 16, 16)] = v >> 16

    def gather_start(b):
        pltpu.async_copy(feat.at[srcb.at[b]], rows.at[b], gsems[b])

    def gather_wait(b):
        pltpu.make_async_copy(feat.at[srcb.at[b]], rows.at[b], gsems[b]).wait()

    def scatter_start(b):
        pltpu.async_copy(rows.at[b], acc.at[dstb.at[b]], ssems[b], add=True)

    def scatter_wait(b):
        pltpu.make_async_copy(rows.at[b], acc.at[dstb.at[b]], ssems[b]).wait()

    for b in range(2):
        unpack(b, b)
        gather_start(b)

    def body(i, carry):
        for b in range(2):
            j = i * 2 + b
            gather_wait(b)
            scatter_start(b)
            nxt = j + 2

            @pl.when(nxt < _NCH)
            def _():
                scatter_wait(b)
                unpack(nxt, b)
                gather_start(b)

        return carry

    lax.fori_loop(0, _NCH // 2, body, 0)
    for b in range(2):
        scatter_wait(b)

    plsc.subcore_barrier()
    pltpu.sync_copy(acc.at[pl.ds(base, _STRIPE)], out.at[c, pl.ds(base, _STRIPE)])


@jax.jit
def _sc_aggregate(feature, pk3):
    mesh = plsc.VectorSubcoreMesh(core_axis_name="c", subcore_axis_name="s")
    k = functools.partial(
        pl.kernel,
        mesh=mesh,
        out_type=jax.ShapeDtypeStruct((_NC, _ACC_N, _D), jnp.float32),
        scratch_types=[
            pltpu.VMEM((_NCH, _CH), jnp.int32),
            pltpu.VMEM((2, _CH), jnp.int32),
            pltpu.VMEM((2, _CH), jnp.int32),
            pltpu.VMEM((2, _CH, _D), jnp.float32),
            pltpu.VMEM_SHARED((_ACC_N, _D), jnp.float32),
            pltpu.SemaphoreType.DMA,
            pltpu.SemaphoreType.DMA,
            pltpu.SemaphoreType.DMA,
            pltpu.SemaphoreType.DMA,
        ],
    )(_sc_body)
    return k(feature, pk3)


def _tc_body(p_ref, wt_ref, b_ref, o_ref):
    x = p_ref[0] + p_ref[1]
    y = jnp.dot(x, wt_ref[...], preferred_element_type=jnp.float32)
    o_ref[...] = jnp.maximum(y + b_ref[...], 0.0)


def _tc_linear(partials, Wt, b2):
    bm = 1000
    return pl.pallas_call(
        _tc_body,
        grid=(_N_NODES // bm,),
        in_specs=[
            pl.BlockSpec((_NC, bm, _D), lambda i: (0, i, 0)),
            pl.BlockSpec((_D, _D), lambda i: (0, 0)),
            pl.BlockSpec((1, _D), lambda i: (0, 0)),
        ],
        out_specs=pl.BlockSpec((bm, _D), lambda i: (i, 0)),
        out_shape=jax.ShapeDtypeStruct((_N_NODES, _D), jnp.float32),
    )(partials, Wt, b2)


def kernel(feature, edge_index, W, b):
    src = edge_index[0].astype(jnp.int32)
    dst = edge_index[1].astype(jnp.int32)
    packed = src | (dst << 16)
    pad = _E_PAD - packed.shape[0]
    ar = jnp.arange(pad, dtype=jnp.int32)
    pad_src = ar % _N_NODES
    pad_dst = _DUMMY + (ar % (_ACC_N - _N_NODES))
    packed = jnp.concatenate([packed, pad_src | (pad_dst << 16)])
    pk3 = packed.reshape(_NW, _NCH, _CH)
    partials = _sc_aggregate(feature, pk3)
    return _tc_linear(partials, W.T, b.reshape(1, _D))
